# baseline (speedup 1.0000x reference)
import functools
import math

import jax
import jax.numpy as jnp
from jax import lax
from jax.experimental import pallas as pl
from jax.experimental.pallas import tpu as pltpu
from jax.experimental.pallas import tpu_sc as plsc

D = 64
WIDE = 128
SCALE = math.sqrt(D)
CHUNK = 128
NBUF = 4
NTB = 2
VBLK = 4096
LANES = 16


def _widen_kernel(tt_ref, out_ref):
  out_ref[:, 0:D] = tt_ref[...].T * SCALE
  out_ref[:, D:WIDE] = jnp.zeros((VBLK, WIDE - D), jnp.float32)


def _widen(table_t):
  vocab = table_t.shape[1]
  grid = (pl.cdiv(vocab, VBLK),)
  return pl.pallas_call(
      _widen_kernel,
      out_shape=jax.ShapeDtypeStruct((vocab, WIDE), jnp.float32),
      grid=grid,
      in_specs=[pl.BlockSpec((D, VBLK), lambda i: (0, i))],
      out_specs=pl.BlockSpec((VBLK, WIDE), lambda i: (i, 0)),
  )(table_t)


def _sc_kernel(npos, nblk, chunks_per_worker, nc, ns):
  ngroups = chunks_per_worker // NBUF
  mesh = plsc.VectorSubcoreMesh(core_axis_name="c", subcore_axis_name="s")

  scratch = [pltpu.VMEM((chunks_per_worker, CHUNK), jnp.int32)]
  scratch += [pltpu.VMEM((CHUNK, WIDE), jnp.float32) for _ in range(NBUF)]
  scratch += [pltpu.VMEM((8, 8, CHUNK), jnp.float32) for _ in range(NTB)]
  scratch += [pltpu.SemaphoreType.DMA for _ in range(NBUF + NTB)]

  @functools.partial(
      pl.kernel,
      out_type=jax.ShapeDtypeStruct((npos, 8, nblk, 8, CHUNK), jnp.float32),
      mesh=mesh,
      scratch_types=scratch,
      compiler_params=pltpu.CompilerParams(needs_layout_passes=False),
  )
  def k(idx_hbm, table_hbm, out_hbm, idx_v, *rest):
    gbuf = rest[:NBUF]
    tbuf = rest[NBUF:NBUF + NTB]
    gsem = rest[NBUF + NTB:2 * NBUF + NTB]
    ssem = rest[2 * NBUF + NTB:]
    wid = lax.axis_index("s") * nc + lax.axis_index("c")
    base = wid * chunks_per_worker

    pltpu.sync_copy(idx_hbm.at[pl.ds(base, chunks_per_worker)], idx_v)

    iotas = [lax.iota(jnp.int32, LANES) + (kk * LANES)
             for kk in range(CHUNK // LANES)]

    def gather_start(j, b):
      pltpu.async_copy(table_hbm.at[idx_v.at[j]], gbuf[b], gsem[b])

    def gather_wait(j, b):
      pltpu.make_async_copy(table_hbm.at[idx_v.at[j]], gbuf[b],
                            gsem[b]).wait()

    def out_slice(j):
      c = base + j
      p = lax.div(c, nblk)
      jb = lax.rem(c, nblk)
      return out_hbm.at[p, :, jb, :, :]

    def scatter_start(j, t):
      pltpu.async_copy(tbuf[t], out_slice(j), ssem[t])

    def scatter_wait(j, t):
      pltpu.make_async_copy(tbuf[t], out_slice(j), ssem[t]).wait()

    def transpose(b, t):
      src = gbuf[b]
      dst = tbuf[t]

      @pl.loop(0, 8)
      def _(fh):
        for fl in range(8):
          colv = jnp.zeros((LANES,), jnp.int32) + (fh * 8 + fl)
          for kk in range(CHUNK // LANES):
            v = plsc.load_gather(src, [iotas[kk], colv])
            dst[fh, fl, pl.ds(kk * LANES, LANES)] = v

    def process(j, b, t, wait_scatter, start_gather):
      gather_wait(j, b)
      if wait_scatter:
        scatter_wait(j - NTB, t)
      transpose(b, t)
      scatter_start(j, t)
      if start_gather:
        gather_start(j + NBUF, b)

    for b in range(NBUF):
      gather_start(b, b)

    for b in range(NBUF):
      process(b, b, b % NTB, b >= NTB, True)

    @pl.loop(1, ngroups - 1)
    def _(g):
      for b in range(NBUF):
        j = g * NBUF + b
        process(j, b, b % NTB, True, True)

    for b in range(NBUF):
      j = (ngroups - 1) * NBUF + b
      process(j, b, b % NTB, True, False)

    for t in range(NTB):
      j = chunks_per_worker - NTB + t
      scatter_wait(j, j % NTB)

  return k


def kernel(x, table):
  xs, ts = x.shape, table.shape
  npos, nbatch = xs[1], xs[0]
  nblk = nbatch // CHUNK
  b_total = nbatch * npos
  info = plsc.get_sparse_core_info()
  nw = info.num_cores * info.num_subcores
  chunks_per_worker = b_total // (CHUNK * nw)
  idx = jnp.reshape(x.T.astype(jnp.int32), (b_total // CHUNK, CHUNK))
  twide = _widen(table.T)
  k = _sc_kernel(npos, nblk, chunks_per_worker, info.num_cores,
                 info.num_subcores)
  out4 = k(idx, twide)
  res = jnp.transpose(out4, (2, 4, 0, 1, 3))
  return jnp.reshape(res, (nbatch, npos, ts[1]))


# speedup vs baseline: 2.7619x; 2.7619x over previous
import functools
import math

import jax
import jax.numpy as jnp
from jax import lax
from jax.experimental import pallas as pl
from jax.experimental.pallas import tpu as pltpu
from jax.experimental.pallas import tpu_sc as plsc

D = 64
WIDE = 128
SCALE = math.sqrt(D)
CHUNK = 128
NBUF = 4
NTB = 2
VBLK = 4096
LANES = 16


def _widen_kernel(tt_ref, out_ref):
  out_ref[:, 0:D] = tt_ref[...].T * SCALE
  out_ref[:, D:WIDE] = jnp.zeros((VBLK, WIDE - D), jnp.float32)


def _widen(table_t):
  vocab = table_t.shape[1]
  grid = (pl.cdiv(vocab, VBLK),)
  return pl.pallas_call(
      _widen_kernel,
      out_shape=jax.ShapeDtypeStruct((vocab, WIDE), jnp.float32),
      grid=grid,
      in_specs=[pl.BlockSpec((D, VBLK), lambda i: (0, i))],
      out_specs=pl.BlockSpec((VBLK, WIDE), lambda i: (i, 0)),
  )(table_t)


def _sc_kernel(npos, nblk, chunks_per_worker, nc, ns):
  ngroups = chunks_per_worker // NBUF
  mesh = plsc.VectorSubcoreMesh(core_axis_name="c", subcore_axis_name="s")

  scratch = [pltpu.VMEM((chunks_per_worker, CHUNK), jnp.int32)]
  scratch += [pltpu.VMEM((CHUNK, WIDE), jnp.float32) for _ in range(NBUF)]
  scratch += [pltpu.VMEM((8, 8, CHUNK), jnp.float32) for _ in range(NTB)]
  scratch += [pltpu.SemaphoreType.DMA for _ in range(NBUF + NTB)]

  @functools.partial(
      pl.kernel,
      out_type=jax.ShapeDtypeStruct((npos, 8, nblk, 8, CHUNK), jnp.float32),
      mesh=mesh,
      scratch_types=scratch,
      compiler_params=pltpu.CompilerParams(needs_layout_passes=False),
  )
  def k(idx_hbm, table_hbm, out_hbm, idx_v, *rest):
    gbuf = rest[:NBUF]
    tbuf = rest[NBUF:NBUF + NTB]
    gsem = rest[NBUF + NTB:2 * NBUF + NTB]
    ssem = rest[2 * NBUF + NTB:]
    wid = lax.axis_index("s") * nc + lax.axis_index("c")
    base = wid * chunks_per_worker

    pltpu.sync_copy(idx_hbm.at[pl.ds(base, chunks_per_worker)], idx_v)

    iotas = [lax.iota(jnp.int32, LANES) + (kk * LANES)
             for kk in range(CHUNK // LANES)]

    def gather_start(j, b):
      pltpu.async_copy(table_hbm.at[idx_v.at[j]], gbuf[b], gsem[b])

    def gather_wait(j, b):
      pltpu.make_async_copy(table_hbm.at[idx_v.at[j]], gbuf[b],
                            gsem[b]).wait()

    def out_slice(j):
      c = base + j
      p = lax.div(c, nblk)
      jb = lax.rem(c, nblk)
      return out_hbm.at[p, :, jb, :, :]

    def scatter_start(j, t):
      pltpu.async_copy(tbuf[t], out_slice(j), ssem[t])

    def scatter_wait(j, t):
      pltpu.make_async_copy(tbuf[t], out_slice(j), ssem[t]).wait()

    def transpose(b, t):
      src = gbuf[b]
      dst = tbuf[t]

      @functools.partial(plsc.parallel_loop, 0, D, unroll=2)
      def _(f):
        colv = jnp.zeros((LANES,), jnp.int32) + f
        fh = lax.div(f, 8)
        fl = lax.rem(f, 8)
        for kk in range(CHUNK // LANES):
          v = plsc.load_gather(src, [iotas[kk], colv])
          dst[fh, fl, pl.ds(kk * LANES, LANES)] = v

    def process(j, b, t, wait_scatter, start_gather):
      gather_wait(j, b)
      if wait_scatter:
        scatter_wait(j - NTB, t)
      transpose(b, t)
      scatter_start(j, t)
      if start_gather:
        gather_start(j + NBUF, b)

    for b in range(NBUF):
      gather_start(b, b)

    for b in range(NBUF):
      process(b, b, b % NTB, b >= NTB, True)

    @pl.loop(1, ngroups - 1)
    def _(g):
      for b in range(NBUF):
        j = g * NBUF + b
        process(j, b, b % NTB, True, True)

    for b in range(NBUF):
      j = (ngroups - 1) * NBUF + b
      process(j, b, b % NTB, True, False)

    for t in range(NTB):
      j = chunks_per_worker - NTB + t
      scatter_wait(j, j % NTB)

  return k


def kernel(x, table):
  xs, ts = x.shape, table.shape
  npos, nbatch = xs[1], xs[0]
  nblk = nbatch // CHUNK
  b_total = nbatch * npos
  info = plsc.get_sparse_core_info()
  nw = info.num_cores * info.num_subcores
  chunks_per_worker = b_total // (CHUNK * nw)
  idx = jnp.reshape(x.T.astype(jnp.int32), (b_total // CHUNK, CHUNK))
  twide = _widen(table.T)
  k = _sc_kernel(npos, nblk, chunks_per_worker, info.num_cores,
                 info.num_subcores)
  out4 = k(idx, twide)
  res = jnp.transpose(out4, (2, 4, 0, 1, 3))
  return jnp.reshape(res, (nbatch, npos, ts[1]))
